# baseline (speedup 1.0000x reference)
import functools

import jax
import jax.numpy as jnp
from jax import lax
from jax.experimental import pallas as pl
from jax.experimental.pallas import tpu as pltpu
from jax.experimental.pallas import tpu_sc as plsc

N_LAYERS = 6
VOCAB = 100000
HIDDEN = 64
B = 1024
L = 50
TOK = B * L
NW = 32
PER_W = TOK // NW
ROWS_W = B // NW
CHUNK_ROWS = 16
CHUNK = CHUNK_ROWS * L
NCHUNK = PER_W // CHUNK
NBUF = 2
NSTEP = N_LAYERS * NCHUNK


def _emb_body(ids_hbm, tab_hbm, out_hbm, idx_v, rows_v, gsems, wsems):
    wid = lax.axis_index("s") * 2 + lax.axis_index("c")
    row_base = wid * ROWS_W
    pltpu.sync_copy(ids_hbm.at[wid], idx_v)

    gathers = [None] * NBUF
    writes = [None] * NBUF

    def step_of(s):
        return s // NCHUNK, s % NCHUNK

    for s in range(NSTEP + 1):
        if s < NSTEP:
            b = s % NBUF
            if writes[b] is not None:
                for w in writes[b]:
                    w.wait()
            layer, c = step_of(s)
            gathers[b] = pltpu.async_copy(
                tab_hbm.at[layer].at[idx_v.at[c]], rows_v.at[b], gsems[b])
        if s >= 1:
            pb = (s - 1) % NBUF
            layer, c = step_of(s - 1)
            gathers[pb].wait()
            ws = []
            for r in range(CHUNK_ROWS):
                gb = row_base + c * CHUNK_ROWS + r
                src = rows_v.at[pb, pl.ds(r * L, L)]
                ws.append(pltpu.async_copy(
                    src, out_hbm.at[layer, gb], wsems[pb]))
                ws.append(pltpu.async_copy(
                    src, out_hbm.at[2 * N_LAYERS - 1 - layer, gb], wsems[pb]))
            writes[pb] = ws

    for b in range(NBUF):
        if writes[b] is not None:
            for w in writes[b]:
                w.wait()


@functools.partial(
    pl.kernel,
    mesh=plsc.VectorSubcoreMesh(core_axis_name="c", subcore_axis_name="s"),
    compiler_params=pltpu.CompilerParams(use_tc_tiling_on_sc=False),
    out_type=jax.ShapeDtypeStruct((2 * N_LAYERS, B, L, HIDDEN), jnp.float32),
    scratch_types=[
        pltpu.VMEM((NCHUNK, CHUNK), jnp.int32),
        pltpu.VMEM((NBUF, CHUNK, HIDDEN), jnp.float32),
        [pltpu.SemaphoreType.DMA] * NBUF,
        [pltpu.SemaphoreType.DMA] * NBUF,
    ],
)
def _emb_kernel(ids_hbm, tab_hbm, out_hbm, idx_v, rows_v, gsems, wsems):
    _emb_body(ids_hbm, tab_hbm, out_hbm, idx_v, rows_v, gsems, wsems)


def kernel(input_ids, tables):
    ids = input_ids.reshape(NW, NCHUNK, CHUNK)
    return _emb_kernel(ids, tables)


# speedup vs baseline: 1.2857x; 1.2857x over previous
import functools

import jax
import jax.numpy as jnp
from jax import lax
from jax.experimental import pallas as pl
from jax.experimental.pallas import tpu as pltpu
from jax.experimental.pallas import tpu_sc as plsc

N_LAYERS = 6
VOCAB = 100000
HIDDEN = 64
B = 1024
L = 50
NW = 32
H_PER_W = HIDDEN // NW
P_CHUNK = 2
NCHUNK = L // P_CHUNK
GRP = B // 16


def _emb_body(ids_hbm, tab_hbm, out_hbm, row_v, idx_v, out_v):
    wid = lax.axis_index("s") * 2 + lax.axis_index("c")

    for hj in range(H_PER_W):
        h = wid * H_PER_W + hj
        for layer in range(N_LAYERS):
            pltpu.sync_copy(tab_hbm.at[layer, h], row_v)

            def chunk_body(c, carry):
                pltpu.sync_copy(ids_hbm.at[pl.ds(c * P_CHUNK, P_CHUNK)],
                                idx_v)

                for p in range(P_CHUNK):
                    def gather_body(j, cc, p=p):
                        idx = idx_v[p, pl.ds(j * 16, 16)]
                        out_v[p, pl.ds(j * 16, 16)] = plsc.load_gather(
                            row_v, [idx])
                        return cc

                    lax.fori_loop(0, GRP, gather_body, 0)

                for p in range(P_CHUNK):
                    pos = c * P_CHUNK + p
                    pltpu.sync_copy(out_v.at[p], out_hbm.at[layer, pos, h])
                    pltpu.sync_copy(
                        out_v.at[p],
                        out_hbm.at[2 * N_LAYERS - 1 - layer, pos, h])
                return carry

            lax.fori_loop(0, NCHUNK, chunk_body, 0)


@functools.partial(
    pl.kernel,
    compiler_params=pltpu.CompilerParams(needs_layout_passes=False),
    mesh=plsc.VectorSubcoreMesh(core_axis_name="c", subcore_axis_name="s"),
    out_type=jax.ShapeDtypeStruct((2 * N_LAYERS, L, HIDDEN, B), jnp.float32),
    scratch_types=[
        pltpu.VMEM((VOCAB,), jnp.float32),
        pltpu.VMEM((P_CHUNK, B), jnp.int32),
        pltpu.VMEM((P_CHUNK, B), jnp.float32),
    ],
)
def _emb_kernel(ids_hbm, tab_hbm, out_hbm, row_v, idx_v, out_v):
    _emb_body(ids_hbm, tab_hbm, out_hbm, row_v, idx_v, out_v)


def kernel(input_ids, tables):
    ids_t = jnp.transpose(input_ids, (1, 0))
    tab_t = jnp.transpose(tables, (0, 2, 1))
    out = _emb_kernel(ids_t, tab_t)
    return jnp.transpose(out, (0, 3, 1, 2))


# speedup vs baseline: 1.7886x; 1.3911x over previous
import functools

import jax
import jax.numpy as jnp
from jax import lax
from jax.experimental import pallas as pl
from jax.experimental.pallas import tpu as pltpu
from jax.experimental.pallas import tpu_sc as plsc

N_LAYERS = 6
VOCAB = 100000
HIDDEN = 64
B = 1024
L = 50
NW = 32
H_PER_W = HIDDEN // NW
P_CHUNK = 5
NCHUNK = L // P_CHUNK
NPAIR = NCHUNK // 2
GRP = B // 16
MIR = 2 * N_LAYERS - 1


def _emb_body(ids_hbm, tab_hbm, out_hbm, row_v, idx_v, out_v0, out_v1,
              rsem, isems, wsems):
    out_bufs = (out_v0, out_v1)
    wid = lax.axis_index("s") * 2 + lax.axis_index("c")

    jobs = [(l, hj) for hj in range(H_PER_W) for l in range(N_LAYERS)]

    def idx_copy(c, bsel):
        return pltpu.make_async_copy(
            ids_hbm.at[pl.ds(c * (P_CHUNK * B), P_CHUNK * B)],
            idx_v.at[bsel], isems[bsel])

    def write_copies(layer, h, c, bsel):
        pos = c * P_CHUNK
        return (
            pltpu.make_async_copy(
                out_bufs[bsel],
                out_hbm.at[layer, pl.ds(pos, P_CHUNK), h],
                wsems[2 * bsel]),
            pltpu.make_async_copy(
                out_bufs[bsel],
                out_hbm.at[MIR - layer, pl.ds(pos, P_CHUNK), h],
                wsems[2 * bsel + 1]),
        )

    def gather_chunk(bsel):
        for p in range(P_CHUNK):
            def gather_body(j, cc, p=p):
                idx = idx_v[bsel, pl.ds(p * B + j * 16, 16)]
                out_bufs[bsel][p, pl.ds(j * 16, 16)] = plsc.load_gather(
                    row_v, [idx])
                return cc

            lax.fori_loop(0, GRP, gather_body, 0, unroll=8)

    l0, hj0 = jobs[0]
    pltpu.sync_copy(tab_hbm.at[l0, wid * H_PER_W + hj0], row_v)

    for jj, (layer, hj) in enumerate(jobs):
        h = wid * H_PER_W + hj

        idx_copy(0, 0).start()
        idx_copy(1, 1).start()

        def pair_body(t, carry, layer=layer, h=h):
            for bsel in range(2):
                c = 2 * t + bsel
                idx_copy(c, bsel).wait()

                @pl.when(c >= 2)
                def _():
                    w0, w1 = write_copies(layer, h, c - 2, bsel)
                    w0.wait()
                    w1.wait()

                gather_chunk(bsel)

                @pl.when(c + 2 < NCHUNK)
                def _():
                    idx_copy(c + 2, bsel).start()

                w0, w1 = write_copies(layer, h, c, bsel)
                w0.start()
                w1.start()
            return carry

        lax.fori_loop(0, NPAIR, pair_body, 0)

        if jj + 1 < len(jobs):
            ln, hjn = jobs[jj + 1]
            pltpu.make_async_copy(
                tab_hbm.at[ln, wid * H_PER_W + hjn], row_v, rsem).start()

        for c in (NCHUNK - 2, NCHUNK - 1):
            w0, w1 = write_copies(layer, h, c, c % 2)
            w0.wait()
            w1.wait()

        if jj + 1 < len(jobs):
            pltpu.make_async_copy(
                tab_hbm.at[0, 0], row_v, rsem).wait()


@functools.partial(
    pl.kernel,
    compiler_params=pltpu.CompilerParams(needs_layout_passes=False),
    mesh=plsc.VectorSubcoreMesh(core_axis_name="c", subcore_axis_name="s"),
    out_type=jax.ShapeDtypeStruct((2 * N_LAYERS, L, HIDDEN, B), jnp.float32),
    scratch_types=[
        pltpu.VMEM((VOCAB,), jnp.float32),
        pltpu.VMEM((2, P_CHUNK * B), jnp.int32),
        pltpu.VMEM((P_CHUNK, B), jnp.float32),
        pltpu.VMEM((P_CHUNK, B), jnp.float32),
        pltpu.SemaphoreType.DMA,
        [pltpu.SemaphoreType.DMA] * 2,
        [pltpu.SemaphoreType.DMA] * 4,
    ],
)
def _emb_kernel(ids_hbm, tab_hbm, out_hbm, row_v, idx_v, out_v0, out_v1,
                rsem, isems, wsems):
    _emb_body(ids_hbm, tab_hbm, out_hbm, row_v, idx_v, out_v0, out_v1,
              rsem, isems, wsems)


def kernel(input_ids, tables):
    ids_t = jnp.transpose(input_ids, (1, 0)).reshape(L * B)
    tab_t = jnp.transpose(tables, (0, 2, 1))
    out = _emb_kernel(ids_t, tab_t)
    return jnp.transpose(out, (0, 3, 1, 2))


# speedup vs baseline: 3.2057x; 1.7923x over previous
import functools

import jax
import jax.numpy as jnp
from jax import lax
from jax.experimental import pallas as pl
from jax.experimental.pallas import tpu as pltpu
from jax.experimental.pallas import tpu_sc as plsc

N_LAYERS = 6
VOCAB = 100000
HIDDEN = 64
B = 1024
L = 50
NW = 32
H_PER_W = HIDDEN // NW
P_CHUNK = 5
NCHUNK = L // P_CHUNK
NPAIR = NCHUNK // 2
GRP = B // 16
MIR = 2 * N_LAYERS - 1


def _emb_body(ids_hbm, tab_hbm, out_hbm, row_v, idx_v, out_v0, out_v1,
              rsem, isems, wsems):
    out_bufs = (out_v0, out_v1)
    wid = lax.axis_index("s") * 2 + lax.axis_index("c")

    jobs = [(l, hj) for hj in range(H_PER_W) for l in range(N_LAYERS)]

    def idx_copy(c, bsel):
        return pltpu.make_async_copy(
            ids_hbm.at[pl.ds(c * (P_CHUNK * B), P_CHUNK * B)],
            idx_v.at[bsel], isems[bsel])

    def write_copies(layer, h, c, bsel):
        pos = c * P_CHUNK
        return (
            pltpu.make_async_copy(
                out_bufs[bsel],
                out_hbm.at[layer, pl.ds(pos, P_CHUNK), h],
                wsems[2 * bsel]),
            pltpu.make_async_copy(
                out_bufs[bsel],
                out_hbm.at[MIR - layer, pl.ds(pos, P_CHUNK), h],
                wsems[2 * bsel + 1]),
        )

    GB = 4

    def gather_chunk(bsel):
        for p in range(P_CHUNK):
            def gather_body(j, cc, p=p):
                idxs = [idx_v[bsel, pl.ds(p * B + (j * GB + k) * 16, 16)]
                        for k in range(GB)]
                vals = [plsc.load_gather(row_v, [ix]) for ix in idxs]
                for k, v in enumerate(vals):
                    out_bufs[bsel][p, pl.ds((j * GB + k) * 16, 16)] = v
                return cc

            lax.fori_loop(0, GRP // GB, gather_body, 0, unroll=2)

    l0, hj0 = jobs[0]
    pltpu.sync_copy(tab_hbm.at[l0, wid * H_PER_W + hj0], row_v)

    for jj, (layer, hj) in enumerate(jobs):
        h = wid * H_PER_W + hj

        idx_copy(0, 0).start()
        idx_copy(1, 1).start()

        def pair_body(t, carry, layer=layer, h=h):
            for bsel in range(2):
                c = 2 * t + bsel
                idx_copy(c, bsel).wait()

                @pl.when(c >= 2)
                def _():
                    w0, w1 = write_copies(layer, h, c - 2, bsel)
                    w0.wait()
                    w1.wait()

                gather_chunk(bsel)

                @pl.when(c + 2 < NCHUNK)
                def _():
                    idx_copy(c + 2, bsel).start()

                w0, w1 = write_copies(layer, h, c, bsel)
                w0.start()
                w1.start()
            return carry

        lax.fori_loop(0, NPAIR, pair_body, 0)

        if jj + 1 < len(jobs):
            ln, hjn = jobs[jj + 1]
            pltpu.make_async_copy(
                tab_hbm.at[ln, wid * H_PER_W + hjn], row_v, rsem).start()

        for c in (NCHUNK - 2, NCHUNK - 1):
            w0, w1 = write_copies(layer, h, c, c % 2)
            w0.wait()
            w1.wait()

        if jj + 1 < len(jobs):
            pltpu.make_async_copy(
                tab_hbm.at[0, 0], row_v, rsem).wait()


@functools.partial(
    pl.kernel,
    compiler_params=pltpu.CompilerParams(needs_layout_passes=False),
    mesh=plsc.VectorSubcoreMesh(core_axis_name="c", subcore_axis_name="s"),
    out_type=jax.ShapeDtypeStruct((2 * N_LAYERS, L, HIDDEN, B), jnp.float32),
    scratch_types=[
        pltpu.VMEM((VOCAB,), jnp.float32),
        pltpu.VMEM((2, P_CHUNK * B), jnp.int32),
        pltpu.VMEM((P_CHUNK, B), jnp.float32),
        pltpu.VMEM((P_CHUNK, B), jnp.float32),
        pltpu.SemaphoreType.DMA,
        [pltpu.SemaphoreType.DMA] * 2,
        [pltpu.SemaphoreType.DMA] * 4,
    ],
)
def _emb_kernel(ids_hbm, tab_hbm, out_hbm, row_v, idx_v, out_v0, out_v1,
                rsem, isems, wsems):
    _emb_body(ids_hbm, tab_hbm, out_hbm, row_v, idx_v, out_v0, out_v1,
              rsem, isems, wsems)


def kernel(input_ids, tables):
    ids_t = jnp.transpose(input_ids, (1, 0)).reshape(L * B)
    tab_t = jnp.transpose(tables, (0, 2, 1))
    out = _emb_kernel(ids_t, tab_t)
    return jnp.transpose(out, (0, 3, 1, 2))


# speedup vs baseline: 3.4479x; 1.0755x over previous
import functools

import jax
import jax.numpy as jnp
from jax import lax
from jax.experimental import pallas as pl
from jax.experimental.pallas import tpu as pltpu
from jax.experimental.pallas import tpu_sc as plsc

N_LAYERS = 6
VOCAB = 100000
HIDDEN = 64
B = 1024
L = 50
NW = 32
H_PER_W = HIDDEN // NW
P_CHUNK = 5
NCHUNK = L // P_CHUNK
NPAIR = NCHUNK // 2
GRP = B // 16
MIR = 2 * N_LAYERS - 1


def _emb_body(ids_hbm, tab_hbm, out_hbm, row_v, idx_v, out_v0, out_v1,
              rsem, isems, wsems):
    out_bufs = (out_v0, out_v1)
    wid = lax.axis_index("s") * 2 + lax.axis_index("c")

    jobs = [(l, hj) for hj in range(H_PER_W) for l in range(N_LAYERS)]

    def idx_copy(c, bsel):
        return pltpu.make_async_copy(
            ids_hbm.at[pl.ds(c * (P_CHUNK * B), P_CHUNK * B)],
            idx_v.at[bsel], isems[bsel])

    def write_copies(layer, h, c, bsel):
        pos = c * P_CHUNK
        return (
            pltpu.make_async_copy(
                out_bufs[bsel],
                out_hbm.at[layer, pl.ds(pos, P_CHUNK), h],
                wsems[2 * bsel]),
            pltpu.make_async_copy(
                out_bufs[bsel],
                out_hbm.at[MIR - layer, pl.ds(pos, P_CHUNK), h],
                wsems[2 * bsel + 1]),
        )

    GB = 4

    def gather_chunk(bsel):
        for p in range(P_CHUNK):
            @plsc.parallel_loop(0, GRP // GB, unroll=2)
            def _(j, p=p):
                idxs = [idx_v[bsel, pl.ds(p * B + (j * GB + k) * 16, 16)]
                        for k in range(GB)]
                vals = [plsc.load_gather(row_v, [ix]) for ix in idxs]
                for k, v in enumerate(vals):
                    out_bufs[bsel][p, pl.ds((j * GB + k) * 16, 16)] = v

    l0, hj0 = jobs[0]
    pltpu.sync_copy(tab_hbm.at[l0, wid * H_PER_W + hj0], row_v)

    for jj, (layer, hj) in enumerate(jobs):
        h = wid * H_PER_W + hj

        idx_copy(0, 0).start()
        idx_copy(1, 1).start()

        def pair_body(t, carry, layer=layer, h=h):
            for bsel in range(2):
                c = 2 * t + bsel
                idx_copy(c, bsel).wait()

                @pl.when(c >= 2)
                def _():
                    w0, w1 = write_copies(layer, h, c - 2, bsel)
                    w0.wait()
                    w1.wait()

                gather_chunk(bsel)

                @pl.when(c + 2 < NCHUNK)
                def _():
                    idx_copy(c + 2, bsel).start()

                w0, w1 = write_copies(layer, h, c, bsel)
                w0.start()
                w1.start()
            return carry

        lax.fori_loop(0, NPAIR, pair_body, 0)

        if jj + 1 < len(jobs):
            ln, hjn = jobs[jj + 1]
            pltpu.make_async_copy(
                tab_hbm.at[ln, wid * H_PER_W + hjn], row_v, rsem).start()

        for c in (NCHUNK - 2, NCHUNK - 1):
            w0, w1 = write_copies(layer, h, c, c % 2)
            w0.wait()
            w1.wait()

        if jj + 1 < len(jobs):
            pltpu.make_async_copy(
                tab_hbm.at[0, 0], row_v, rsem).wait()


@functools.partial(
    pl.kernel,
    compiler_params=pltpu.CompilerParams(needs_layout_passes=False),
    mesh=plsc.VectorSubcoreMesh(core_axis_name="c", subcore_axis_name="s"),
    out_type=jax.ShapeDtypeStruct((2 * N_LAYERS, L, HIDDEN, B), jnp.float32),
    scratch_types=[
        pltpu.VMEM((VOCAB,), jnp.float32),
        pltpu.VMEM((2, P_CHUNK * B), jnp.int32),
        pltpu.VMEM((P_CHUNK, B), jnp.float32),
        pltpu.VMEM((P_CHUNK, B), jnp.float32),
        pltpu.SemaphoreType.DMA,
        [pltpu.SemaphoreType.DMA] * 2,
        [pltpu.SemaphoreType.DMA] * 4,
    ],
)
def _emb_kernel(ids_hbm, tab_hbm, out_hbm, row_v, idx_v, out_v0, out_v1,
                rsem, isems, wsems):
    _emb_body(ids_hbm, tab_hbm, out_hbm, row_v, idx_v, out_v0, out_v1,
              rsem, isems, wsems)


def kernel(input_ids, tables):
    ids_t = jnp.transpose(input_ids, (1, 0)).reshape(L * B)
    tab_t = jnp.transpose(tables, (0, 2, 1))
    out = _emb_kernel(ids_t, tab_t)
    return jnp.transpose(out, (0, 3, 1, 2))
